# baseline (speedup 1.0000x reference)
import functools

import jax
import jax.numpy as jnp
from jax import lax
from jax.experimental import pallas as pl
from jax.experimental.pallas import tpu as pltpu
from jax.experimental.pallas import tpu_sc as plsc

B, C, T, H, W = 8, 768, 16, 14, 14
N = B * C * T
P = H * W
PAD = H // 7
PV = (H - PAD) * W
VALID = tuple(
    i * W + j for i in range(H - PAD) for j in range(PAD, W - PAD)
)
NVALID = len(VALID)

NW = 32
NPW = N // NW
CHUNK = 256
NCHUNK = NPW // CHUNK
GROUPS = CHUNK // 16

_mesh = plsc.VectorSubcoreMesh(core_axis_name="c", subcore_axis_name="s")


@functools.partial(
    pl.kernel,
    out_type=jax.ShapeDtypeStruct((N,), jnp.float32),
    mesh=_mesh,
    scratch_types=[
        pltpu.VMEM((2, CHUNK, PV), jnp.float32),
        pltpu.VMEM((NPW,), jnp.float32),
        pltpu.SemaphoreType.DMA,
        pltpu.SemaphoreType.DMA,
    ],
    compiler_params=pltpu.CompilerParams(
        use_tc_tiling_on_sc=False, needs_layout_passes=False
    ),
)
def _pool_kernel(x_hbm, out_hbm, buf, outv, sem0, sem1):
    wid = lax.axis_index("s") * 2 + lax.axis_index("c")
    base = wid * NPW
    sems = (sem0, sem1)

    def start_dma(g, b):
        pltpu.async_copy(
            x_hbm.at[pl.ds(base + g * CHUNK, CHUNK), pl.ds(0, PV)],
            buf.at[b],
            sems[b],
        )

    def wait_dma(b):
        pltpu.make_async_copy(
            x_hbm.at[pl.ds(base, CHUNK), pl.ds(0, PV)],
            buf.at[b],
            sems[b],
        ).wait()

    def compute(g, b):
        inv = jnp.float32(1.0 / NVALID)

        def group_body(t, _):
            rows = lax.iota(jnp.int32, 16) + t * 16

            def row_body(i, accs):
                acc0, acc1 = accs
                rowbase = i * W
                for j in range(PAD, W - PAD):
                    cols = jnp.full((16,), rowbase + j, jnp.int32)
                    v = plsc.load_gather(buf.at[b], [rows, cols])
                    if j % 2 == 0:
                        acc0 = acc0 + v
                    else:
                        acc1 = acc1 + v
                return acc0, acc1

            z = jnp.zeros((16,), jnp.float32)
            acc0, acc1 = lax.fori_loop(0, H - PAD, row_body, (z, z))
            acc = acc0 + acc1
            outv[pl.ds(g * CHUNK + t * 16, 16)] = acc * inv
            return 0

        lax.fori_loop(0, GROUPS, group_body, 0)

    start_dma(0, 0)
    start_dma(1, 1)

    def chunk_pair(g2, _):
        for b in (0, 1):
            g = g2 * 2 + b
            wait_dma(b)
            compute(g, b)

            @pl.when(g + 2 < NCHUNK)
            def _():
                start_dma(g + 2, b)

        return 0

    lax.fori_loop(0, NCHUNK // 2, chunk_pair, 0)
    pltpu.sync_copy(outv, out_hbm.at[pl.ds(base, NPW)])


def kernel(x):
    b, c, t, h, w = x.shape
    y = _pool_kernel(x.reshape(N, P))
    return y.reshape(b, c, t)


# speedup vs baseline: 16.6627x; 16.6627x over previous
import functools

import jax
import jax.numpy as jnp
from jax import lax
from jax.experimental import pallas as pl
from jax.experimental.pallas import tpu as pltpu
from jax.experimental.pallas import tpu_sc as plsc

B, C, T, H, W = 8, 768, 16, 14, 14
PAD = H // 7
HV = H - PAD
WV = W - 2 * PAD
NVALID = HV * WV
CC = C // 16

NC, NS = 2, 16
HG = 4
HPG = HV // HG
SLAB = 2
NCHUNK = HPG * (WV // SLAB)

_mesh = plsc.VectorSubcoreMesh(core_axis_name="c", subcore_axis_name="s")


@functools.partial(
    pl.kernel,
    out_type=jax.ShapeDtypeStruct((B, T, C), jnp.float32),
    mesh=_mesh,
    scratch_types=[
        pltpu.VMEM((2, SLAB, T, C), jnp.float32),
        pltpu.VMEM((T, C), jnp.float32),
        pltpu.VMEM_SHARED((NS, T, C), jnp.float32),
        pltpu.SemaphoreType.DMA,
        pltpu.SemaphoreType.DMA,
    ],
)
def _pool_kernel(x_hbm, out_hbm, buf, acc, shared, sem0, sem1):
    cid = lax.axis_index("c")
    sid = lax.axis_index("s")
    b = cid * (NS // HG) + sid // HG
    m = sid % HG
    g = sid // HG
    sems = (sem0, sem1)
    inv = jnp.float32(1.0 / NVALID)

    def chunk_hw(k):
        return 3 * m + k // 5, PAD + SLAB * (k % 5)

    def start_dma(k, p):
        h, w0 = chunk_hw(k)
        pltpu.async_copy(
            x_hbm.at[b, h, pl.ds(w0, SLAB)], buf.at[p], sems[p]
        )

    def wait_dma(p):
        pltpu.make_async_copy(
            x_hbm.at[0, 0, pl.ds(0, SLAB)], buf.at[p], sems[p]
        ).wait()

    def accumulate(p):
        @plsc.parallel_loop(0, T)
        def _(t):
            @plsc.parallel_loop(0, CC)
            def _(cc):
                c0 = cc * 16
                v = buf[p, 0, t, pl.ds(c0, 16)] + buf[p, 1, t, pl.ds(c0, 16)]
                plsc.addupdate(acc.at[t, pl.ds(c0, 16)], v)

    @plsc.parallel_loop(0, T)
    def _(t):
        @plsc.parallel_loop(0, CC)
        def _(cc):
            acc[t, pl.ds(cc * 16, 16)] = jnp.zeros((16,), jnp.float32)

    start_dma(0, 0)
    start_dma(1, 1)
    for k in range(NCHUNK):
        p = k % 2
        wait_dma(p)
        if k + 2 < NCHUNK:
            accumulate(p)
            start_dma(k + 2, p)
        else:
            accumulate(p)

    @plsc.parallel_loop(0, T)
    def _(t):
        @plsc.parallel_loop(0, CC)
        def _(cc):
            c0 = cc * 16
            acc[t, pl.ds(c0, 16)] = acc[t, pl.ds(c0, 16)] * inv

    @pl.when(m != 0)
    def _():
        pltpu.sync_copy(acc, shared.at[sid])

    plsc.subcore_barrier()

    @pl.when(m == 0)
    def _():
        pltpu.sync_copy(shared.at[sid + 1], buf.at[0, 0])
        pltpu.sync_copy(shared.at[sid + 2], buf.at[0, 1])
        pltpu.sync_copy(shared.at[sid + 3], buf.at[1, 0])

        @plsc.parallel_loop(0, T)
        def _(t):
            @plsc.parallel_loop(0, CC)
            def _(cc):
                c0 = cc * 16
                v = (
                    buf[0, 0, t, pl.ds(c0, 16)]
                    + buf[0, 1, t, pl.ds(c0, 16)]
                    + buf[1, 0, t, pl.ds(c0, 16)]
                )
                plsc.addupdate(acc.at[t, pl.ds(c0, 16)], v)

        pltpu.sync_copy(acc, out_hbm.at[b])


def kernel(x):
    xt = jnp.transpose(x, (0, 3, 4, 2, 1))
    y = _pool_kernel(xt)
    return jnp.transpose(y, (0, 2, 1))


# speedup vs baseline: 25.0789x; 1.5051x over previous
import functools

import jax
import jax.numpy as jnp
from jax import lax
from jax.experimental import pallas as pl
from jax.experimental.pallas import tpu as pltpu
from jax.experimental.pallas import tpu_sc as plsc

B, C, T, H, W = 8, 768, 16, 14, 14
PAD = H // 7
HV = H - PAD
WV = W - 2 * PAD
NVALID = HV * WV
POS = T * C // 16

NC, NS = 2, 16
HG = 4
SLAB = 2
NCHUNK = (HV // HG) * (WV // SLAB)
NBUF = 4

_mesh = plsc.VectorSubcoreMesh(core_axis_name="c", subcore_axis_name="s")


@functools.partial(
    pl.kernel,
    out_type=jax.ShapeDtypeStruct((B, T, C), jnp.float32),
    mesh=_mesh,
    scratch_types=[
        pltpu.VMEM((NBUF, SLAB, T, C), jnp.float32),
        pltpu.VMEM((T, C), jnp.float32),
        pltpu.VMEM_SHARED((NS, T, C), jnp.float32),
        pltpu.SemaphoreType.DMA,
        pltpu.SemaphoreType.DMA,
        pltpu.SemaphoreType.DMA,
        pltpu.SemaphoreType.DMA,
    ],
)
def _pool_kernel(x_hbm, out_hbm, buf, acc, shared, sem0, sem1, sem2, sem3):
    cid = lax.axis_index("c")
    sid = lax.axis_index("s")
    b = cid * (NS // HG) + sid // HG
    m = sid % HG
    sems = (sem0, sem1, sem2, sem3)
    inv = jnp.float32(1.0 / NVALID)

    def split(v):
        return v & (T - 1), (v >> 4) * 16

    def start_dma(k, p):
        h = 3 * m + k // 5
        w0 = PAD + SLAB * (k % 5)
        pltpu.async_copy(x_hbm.at[b, h, pl.ds(w0, SLAB)], buf.at[p], sems[p])

    def wait_dma(p):
        pltpu.make_async_copy(
            x_hbm.at[0, 0, pl.ds(0, SLAB)], buf.at[p], sems[p]
        ).wait()

    def accumulate(p):
        @plsc.parallel_loop(0, POS, unroll=8)
        def _(v):
            t, c0 = split(v)
            s = buf[p, 0, t, pl.ds(c0, 16)] + buf[p, 1, t, pl.ds(c0, 16)]
            plsc.addupdate(acc.at[t, pl.ds(c0, 16)], s)

    @plsc.parallel_loop(0, POS, unroll=8)
    def _(v):
        t, c0 = split(v)
        acc[t, pl.ds(c0, 16)] = jnp.zeros((16,), jnp.float32)

    for p in range(NBUF - 1):
        start_dma(p, p)
    for k in range(NCHUNK):
        p = k % NBUF
        wait_dma(p)
        accumulate(p)
        if k + NBUF - 1 < NCHUNK:
            start_dma(k + NBUF - 1, (k + NBUF - 1) % NBUF)

    @plsc.parallel_loop(0, POS, unroll=8)
    def _(v):
        t, c0 = split(v)
        acc[t, pl.ds(c0, 16)] = acc[t, pl.ds(c0, 16)] * inv

    @pl.when(m != 0)
    def _():
        pltpu.sync_copy(acc, shared.at[sid])

    plsc.subcore_barrier()

    @pl.when(m == 0)
    def _():
        pltpu.sync_copy(shared.at[sid + 1], buf.at[0, 0])
        pltpu.sync_copy(shared.at[sid + 2], buf.at[1, 0])
        pltpu.sync_copy(shared.at[sid + 3], buf.at[2, 0])

        @plsc.parallel_loop(0, POS, unroll=8)
        def _(v):
            t, c0 = split(v)
            s = (
                buf[0, 0, t, pl.ds(c0, 16)]
                + buf[1, 0, t, pl.ds(c0, 16)]
                + buf[2, 0, t, pl.ds(c0, 16)]
            )
            plsc.addupdate(acc.at[t, pl.ds(c0, 16)], s)

        pltpu.sync_copy(acc, out_hbm.at[b])


def kernel(x):
    xt = jnp.transpose(x, (0, 3, 4, 2, 1))
    y = _pool_kernel(xt)
    return jnp.transpose(y, (0, 2, 1))


# speedup vs baseline: 25.3423x; 1.0105x over previous
import functools

import jax
import jax.numpy as jnp
from jax import lax
from jax.experimental import pallas as pl
from jax.experimental.pallas import tpu as pltpu
from jax.experimental.pallas import tpu_sc as plsc

B, C, T, H, W = 8, 768, 16, 14, 14
PAD = H // 7
HV = H - PAD
WV = W - 2 * PAD
NVALID = HV * WV
POS = T * C // 16

NC, NS = 2, 16
HG = 4
SLAB = 2
NCHUNK = (HV // HG) * (WV // SLAB)
NBUF = 4
UNROLL = 16
TQ = T // 2

_mesh = plsc.VectorSubcoreMesh(core_axis_name="c", subcore_axis_name="s")


@functools.partial(
    pl.kernel,
    out_type=jax.ShapeDtypeStruct((B, T, C), jnp.float32),
    mesh=_mesh,
    scratch_types=[
        pltpu.VMEM((NBUF, SLAB, T, C), jnp.float32),
        pltpu.VMEM((T, C), jnp.float32),
        pltpu.VMEM_SHARED((NS, T, C), jnp.float32),
        pltpu.SemaphoreType.DMA,
        pltpu.SemaphoreType.DMA,
        pltpu.SemaphoreType.DMA,
        pltpu.SemaphoreType.DMA,
    ],
)
def _pool_kernel(x_hbm, out_hbm, buf, acc, shared, sem0, sem1, sem2, sem3):
    cid = lax.axis_index("c")
    sid = lax.axis_index("s")
    b = cid * (NS // HG) + sid // HG
    m = sid % HG
    sems = (sem0, sem1, sem2, sem3)
    inv = jnp.float32(1.0 / NVALID)

    def split(v):
        return v & (T - 1), (v >> 4) * 16

    def start_dma(k, p):
        h = 3 * m + k // 5
        w0 = PAD + SLAB * (k % 5)
        pltpu.async_copy(x_hbm.at[b, h, pl.ds(w0, SLAB)], buf.at[p], sems[p])

    def wait_dma(p):
        pltpu.make_async_copy(
            x_hbm.at[0, 0, pl.ds(0, SLAB)], buf.at[p], sems[p]
        ).wait()

    def accumulate(p, first):
        @plsc.parallel_loop(0, POS, unroll=UNROLL)
        def _(v):
            t, c0 = split(v)
            s = buf[p, 0, t, pl.ds(c0, 16)] + buf[p, 1, t, pl.ds(c0, 16)]
            if first:
                acc[t, pl.ds(c0, 16)] = s
            else:
                plsc.addupdate(acc.at[t, pl.ds(c0, 16)], s)

    for p in range(NBUF - 1):
        start_dma(p, p)
    for k in range(NCHUNK):
        p = k % NBUF
        wait_dma(p)
        accumulate(p, first=(k == 0))
        if k + NBUF - 1 < NCHUNK:
            start_dma(k + NBUF - 1, (k + NBUF - 1) % NBUF)

    pltpu.sync_copy(acc, shared.at[sid])
    plsc.subcore_barrier()

    @pl.when(m < 2)
    def _():
        base = (sid // HG) * HG
        t0 = m * TQ
        for j in range(HG):
            pltpu.sync_copy(
                shared.at[base + j, pl.ds(t0, TQ)],
                buf.at[j, 0, pl.ds(0, TQ)],
            )

        @plsc.parallel_loop(0, TQ * C // 16, unroll=UNROLL)
        def _(v):
            t, c0 = v & (TQ - 1), (v >> 3) * 16
            s = (
                buf[0, 0, t, pl.ds(c0, 16)] + buf[1, 0, t, pl.ds(c0, 16)]
            ) + (
                buf[2, 0, t, pl.ds(c0, 16)] + buf[3, 0, t, pl.ds(c0, 16)]
            )
            acc[t, pl.ds(c0, 16)] = s * inv

        pltpu.sync_copy(
            acc.at[pl.ds(0, TQ)], out_hbm.at[b, pl.ds(t0, TQ)]
        )


def kernel(x):
    xt = jnp.transpose(x, (0, 3, 4, 2, 1))
    y = _pool_kernel(xt)
    return jnp.transpose(y, (0, 2, 1))
